# baseline (speedup 1.0000x reference)
import functools
import jax
import jax.numpy as jnp
from jax import lax
from jax.experimental import pallas as pl
from jax.experimental.pallas import tpu as pltpu
from jax.experimental.pallas import tpu_sc as plsc

NC, NS, L = 2, 16, 16
NW = NC * NS
EMBED_DIM = 64
SCALE = 8.0

BATCH = 4096
ROW_W = 200
BT = BATCH // NW
NBUF = 2


def _make_kernel():
    mesh = plsc.VectorSubcoreMesh(
        core_axis_name="c", subcore_axis_name="s", num_cores=NC, num_subcores=NS
    )

    @functools.partial(
        pl.kernel,
        out_type=jax.ShapeDtypeStruct(
            (ROW_W, EMBED_DIM // 8, NW, 8, BT), jnp.float32
        ),
        mesh=mesh,
        scratch_types=[
            pltpu.VMEM((ROW_W, BT), jnp.int32),
            pltpu.VMEM((BT, EMBED_DIM), jnp.float32),
            pltpu.VMEM((BT, EMBED_DIM), jnp.float32),
            pltpu.VMEM((EMBED_DIM // 8, 8, BT), jnp.float32),
            pltpu.VMEM((EMBED_DIM // 8, 8, BT), jnp.float32),
            pltpu.SemaphoreType.DMA,
            pltpu.SemaphoreType.DMA,
            pltpu.SemaphoreType.DMA,
            pltpu.SemaphoreType.DMA,
        ],
        compiler_params=pltpu.CompilerParams(use_tc_tiling_on_sc=False, needs_layout_passes=False),
    )
    def emb_kernel(tok_hbm, table_hbm, out_hbm, idx_v, b0, b1, t0, t1, g0, g1, s0, s1):
        wid = lax.axis_index("s") * NC + lax.axis_index("c")
        bufs = (b0, b1)
        tbufs = (t0, t1)
        gsem = (g0, g1)
        ssem = (s0, s1)
        pltpu.sync_copy(tok_hbm.at[:, pl.ds(wid * BT, BT)], idx_v)

        lane = lax.iota(jnp.int32, 16)

        def fire_gather(b, c):
            pltpu.async_copy(
                table_hbm.at[idx_v.at[c]], bufs[b], gsem[b]
            )

        def drain_gather(b):
            pltpu.make_async_copy(
                table_hbm.at[idx_v.at[0]], bufs[b], gsem[b]
            ).wait()

        def transpose_scale(b):
            @pl.loop(0, EMBED_DIM)
            def _(j):
                jt = j // 8
                s = j % 8
                jvec = jnp.zeros((16,), jnp.int32) + j
                for k in range(BT // 16):
                    v = plsc.load_gather(
                        bufs[b], [lane + (k * 16), jvec]
                    )
                    tbufs[b][jt, s, pl.ds(k * 16, 16)] = v * SCALE

        def wait_store(b):
            pltpu.make_async_copy(
                tbufs[b], out_hbm.at[0, :, wid], ssem[b]
            ).wait()

        fire_gather(0, 0)

        @pl.loop(0, ROW_W // NBUF)
        def _(o):
            for b in range(NBUF):
                c = o * NBUF + b
                nb = (b + 1) % NBUF

                @pl.when(c + 1 < ROW_W)
                def _():
                    @pl.when(c >= 1)
                    def _():
                        wait_store(nb)

                    fire_gather(nb, c + 1)

                drain_gather(b)
                transpose_scale(b)
                pltpu.async_copy(tbufs[b], out_hbm.at[c, :, wid], ssem[b])

        wait_store(0)
        wait_store(1)

    return emb_kernel


_emb_kernel = _make_kernel()


@jax.jit
def kernel(tokens, table):
    tok_t = tokens.T.astype(jnp.int32)
    o5 = _emb_kernel(tok_t, table)
    return o5.transpose(2, 4, 0, 1, 3).reshape(BATCH, ROW_W, EMBED_DIM)


# speedup vs baseline: 1.7448x; 1.7448x over previous
import functools
import jax
import jax.numpy as jnp
from jax import lax
from jax.experimental import pallas as pl
from jax.experimental.pallas import tpu as pltpu
from jax.experimental.pallas import tpu_sc as plsc

NC, NS, L = 2, 16, 16
NW = NC * NS
EMBED_DIM = 64
SCALE = 8.0

BATCH = 4096
ROW_W = 200
BT = BATCH // NW
NBUF = 2


def _make_kernel():
    mesh = plsc.VectorSubcoreMesh(
        core_axis_name="c", subcore_axis_name="s", num_cores=NC, num_subcores=NS
    )

    @functools.partial(
        pl.kernel,
        out_type=jax.ShapeDtypeStruct(
            (ROW_W, EMBED_DIM // 8, NW, 8, BT), jnp.float32
        ),
        mesh=mesh,
        scratch_types=[
            pltpu.VMEM((ROW_W, BT), jnp.int32),
            pltpu.VMEM((BT, EMBED_DIM), jnp.float32),
            pltpu.VMEM((BT, EMBED_DIM), jnp.float32),
            pltpu.VMEM((EMBED_DIM // 8, 8, BT + 1), jnp.float32),
            pltpu.VMEM((EMBED_DIM // 8, 8, BT + 1), jnp.float32),
            pltpu.SemaphoreType.DMA,
            pltpu.SemaphoreType.DMA,
            pltpu.SemaphoreType.DMA,
            pltpu.SemaphoreType.DMA,
        ],
        compiler_params=pltpu.CompilerParams(use_tc_tiling_on_sc=False, needs_layout_passes=False),
    )
    def emb_kernel(tok_hbm, table_hbm, out_hbm, idx_v, b0, b1, t0, t1, g0, g1, s0, s1):
        wid = lax.axis_index("s") * NC + lax.axis_index("c")
        bufs = (b0, b1)
        tbufs = (t0, t1)
        gsem = (g0, g1)
        ssem = (s0, s1)
        pltpu.sync_copy(tok_hbm.at[:, pl.ds(wid * BT, BT)], idx_v)

        lane = lax.iota(jnp.int32, 16)

        def fire_gather(b, c):
            pltpu.async_copy(
                table_hbm.at[idx_v.at[c]], bufs[b], gsem[b]
            )

        def drain_gather(b):
            pltpu.make_async_copy(
                table_hbm.at[idx_v.at[0]], bufs[b], gsem[b]
            ).wait()

        jt_c = [(lane + k * 16) // 8 for k in range(EMBED_DIM // 16)]
        s_c = [(lane + k * 16) % 8 for k in range(EMBED_DIM // 16)]

        def transpose_scale(b):
            @pl.loop(0, BT, unroll=8)
            def _(l):
                lvec = jnp.zeros((16,), jnp.int32) + l
                for k in range(EMBED_DIM // 16):
                    v = bufs[b][l, pl.ds(k * 16, 16)]
                    plsc.store_scatter(tbufs[b], [jt_c[k], s_c[k], lvec], v * SCALE)

        def wait_store(b):
            pltpu.make_async_copy(
                tbufs[b].at[:, :, pl.ds(0, BT)], out_hbm.at[0, :, wid], ssem[b]
            ).wait()

        fire_gather(0, 0)

        @pl.loop(0, ROW_W // NBUF)
        def _(o):
            for b in range(NBUF):
                c = o * NBUF + b
                nb = (b + 1) % NBUF

                @pl.when(c + 1 < ROW_W)
                def _():
                    @pl.when(c >= 1)
                    def _():
                        wait_store(nb)

                    fire_gather(nb, c + 1)

                drain_gather(b)
                transpose_scale(b)
                pltpu.async_copy(
                    tbufs[b].at[:, :, pl.ds(0, BT)], out_hbm.at[c, :, wid], ssem[b]
                )

        wait_store(0)
        wait_store(1)

    return emb_kernel


_emb_kernel = _make_kernel()


@jax.jit
def kernel(tokens, table):
    tok_t = tokens.T.astype(jnp.int32)
    o5 = _emb_kernel(tok_t, table)
    return o5.transpose(2, 4, 0, 1, 3).reshape(BATCH, ROW_W, EMBED_DIM)
